# Initial kernel scaffold; baseline (speedup 1.0000x reference)
import jax
import jax.numpy as jnp
from jax.experimental import pallas as pl


def kernel(x, edge_attr, W_edge, b_edge, eps, W1, b1, g1, be1, W2, b2, g_bn, b_bn, edge_index, batch):
    raise NotImplementedError("write your pallas kernel here")


# speedup vs baseline: 1.3358x; 1.3358x over previous
import functools

import jax
import jax.numpy as jnp
from jax import lax
from jax.experimental import pallas as pl
from jax.experimental.pallas import tpu as pltpu
from jax.experimental.pallas import tpu_sc as plsc

NUM_LAYER = 5
EMB = 128
D_EDGE = 16
N_NODES = 10000
N_EDGES = 320000
NUM_GRAPHS = 64

NC = 2
NS = 16
NW = NC * NS

CHUNK = 128
NCHUNK = 80
EW = CHUNK * NCHUNK
E_PAD = EW * NW
NPAD = 10112
ROWS_PER_TILE = NPAD // NS
VEC = 16

@functools.cache
def _make_sc_aggregate():
    mesh = plsc.VectorSubcoreMesh(core_axis_name="c", subcore_axis_name="s",
                                  num_cores=NC, num_subcores=NS)
    return pl.kernel(
        _sc_aggregate_body,
        out_type=jax.ShapeDtypeStruct((NC, NPAD, EMB), jnp.float32),
        mesh=mesh,
        scratch_types=[
            pltpu.VMEM((CHUNK,), jnp.int32),
            pltpu.VMEM((CHUNK,), jnp.int32),
            pltpu.VMEM((CHUNK, EMB), jnp.float32),
            pltpu.VMEM((CHUNK, EMB), jnp.float32),
            pltpu.VMEM((CHUNK, EMB), jnp.float32),
            pltpu.VMEM_SHARED((NPAD, EMB), jnp.float32),
        ],
    )


def _sc_aggregate_body(h_hbm, e_hbm, src_hbm, dst_hbm, out_hbm,
                       srcv, dstv, hv, ev, zv, acc):
    c = lax.axis_index("c")
    s = lax.axis_index("s")
    wid = s * NC + c
    edge_base = wid * EW
    row_base = s * ROWS_PER_TILE

    zero = jnp.zeros((VEC,), jnp.float32)

    def _zrow(r, carry):
        for j in range(EMB // VEC):
            zv[r, pl.ds(j * VEC, VEC)] = zero
        return carry
    lax.fori_loop(0, CHUNK, _zrow, 0, unroll=4)

    for t in range(ROWS_PER_TILE // CHUNK):
        pltpu.sync_copy(zv, acc.at[pl.ds(row_base + t * CHUNK, CHUNK)])
    rem = ROWS_PER_TILE % CHUNK
    pltpu.sync_copy(zv.at[pl.ds(0, rem)],
                    acc.at[pl.ds(row_base + (ROWS_PER_TILE // CHUNK) * CHUNK,
                                 rem)])
    plsc.subcore_barrier()

    def _chunk(k, carry):
        base = edge_base + k * CHUNK
        pltpu.sync_copy(src_hbm.at[pl.ds(base, CHUNK)], srcv)
        pltpu.sync_copy(dst_hbm.at[pl.ds(base, CHUNK)], dstv)
        pltpu.sync_copy(e_hbm.at[pl.ds(base, CHUNK)], ev)
        pltpu.sync_copy(h_hbm.at[srcv], hv)

        def _crow(r, inner):
            for j in range(EMB // VEC):
                sl = pl.ds(j * VEC, VEC)
                hv[r, sl] = jnp.maximum(hv[r, sl] + ev[r, sl], 0.0)
            return inner
        lax.fori_loop(0, CHUNK, _crow, 0, unroll=4)

        pltpu.sync_copy(hv, acc.at[dstv], add=True)
        return carry
    lax.fori_loop(0, NCHUNK, _chunk, 0)

    plsc.subcore_barrier()
    pltpu.sync_copy(acc.at[pl.ds(row_base, ROWS_PER_TILE)],
                    out_hbm.at[c, pl.ds(row_base, ROWS_PER_TILE)])


EBLK = 2048


def _encode_body(attr_ref, w_ref, b_ref, out_ref):
    out_ref[...] = jnp.dot(attr_ref[...], w_ref[...],
                           preferred_element_type=jnp.float32) + b_ref[...]


def _encode(attr_pad, w, b):
    return pl.pallas_call(
        _encode_body,
        grid=(E_PAD // EBLK,),
        in_specs=[
            pl.BlockSpec((EBLK, D_EDGE), lambda i: (i, 0)),
            pl.BlockSpec((D_EDGE, EMB), lambda i: (0, 0)),
            pl.BlockSpec((1, EMB), lambda i: (0, 0)),
        ],
        out_specs=pl.BlockSpec((EBLK, EMB), lambda i: (i, 0)),
        out_shape=jax.ShapeDtypeStruct((E_PAD, EMB), jnp.float32),
    )(attr_pad, w, b.reshape(1, EMB))


NBLK = 400


def _mlp_body(h_ref, a_ref, scale_ref, w1_ref, b1_ref, g1_ref, be1_ref,
              w2_ref, b2_ref, gbn_ref, bbn_ref, out_ref, *, last):
    z = h_ref[...] * scale_ref[...] + a_ref[0] + a_ref[1]
    t = jnp.dot(z, w1_ref[...], preferred_element_type=jnp.float32)
    t = t + b1_ref[...]
    t = jnp.maximum(t * g1_ref[...] + be1_ref[...], 0.0)
    o = jnp.dot(t, w2_ref[...], preferred_element_type=jnp.float32)
    o = o + b2_ref[...]
    o = o * gbn_ref[...] + bbn_ref[...]
    if not last:
        o = jnp.maximum(o, 0.0)
    out_ref[...] = o


def _mlp(h, agg2, scale, w1, b1, g1, be1, w2, b2, gbn, bbn, last):
    body = functools.partial(_mlp_body, last=last)
    row = lambda v: v.reshape(1, -1)
    return pl.pallas_call(
        body,
        grid=(N_NODES // NBLK,),
        in_specs=[
            pl.BlockSpec((NBLK, EMB), lambda i: (i, 0)),
            pl.BlockSpec((NC, NBLK, EMB), lambda i: (0, i, 0)),
            pl.BlockSpec((1, EMB), lambda i: (0, 0)),
            pl.BlockSpec((EMB, 2 * EMB), lambda i: (0, 0)),
            pl.BlockSpec((1, 2 * EMB), lambda i: (0, 0)),
            pl.BlockSpec((1, 2 * EMB), lambda i: (0, 0)),
            pl.BlockSpec((1, 2 * EMB), lambda i: (0, 0)),
            pl.BlockSpec((2 * EMB, EMB), lambda i: (0, 0)),
            pl.BlockSpec((1, EMB), lambda i: (0, 0)),
            pl.BlockSpec((1, EMB), lambda i: (0, 0)),
            pl.BlockSpec((1, EMB), lambda i: (0, 0)),
        ],
        out_specs=pl.BlockSpec((NBLK, EMB), lambda i: (i, 0)),
        out_shape=jax.ShapeDtypeStruct((N_NODES, EMB), jnp.float32),
    )(h, agg2, scale, w1, row(b1), row(g1), row(be1), w2, row(b2),
      row(gbn), row(bbn))


PBLK = 2000


def _pool_body(h_ref, batch_ref, out_ref, sums_ref, cnts_ref):
    i = pl.program_id(0)
    gids = lax.broadcasted_iota(jnp.int32, (NUM_GRAPHS, PBLK), 0)
    oh = (gids == batch_ref[0]).astype(jnp.float32)
    psum = jnp.dot(oh, h_ref[...], preferred_element_type=jnp.float32)
    pcnt = jnp.broadcast_to(jnp.sum(oh, axis=1, keepdims=True),
                            (NUM_GRAPHS, EMB))

    @pl.when(i == 0)
    def _init():
        sums_ref[...] = psum
        cnts_ref[...] = pcnt

    @pl.when(i > 0)
    def _accum():
        sums_ref[...] += psum
        cnts_ref[...] += pcnt

    @pl.when(i == pl.num_programs(0) - 1)
    def _final():
        out_ref[...] = sums_ref[...] / jnp.maximum(cnts_ref[...], 1.0)


def _pool(h, batch2d):
    return pl.pallas_call(
        _pool_body,
        grid=(N_NODES // PBLK,),
        in_specs=[
            pl.BlockSpec((PBLK, EMB), lambda i: (i, 0)),
            pl.BlockSpec((1, 1, PBLK), lambda i: (i, 0, 0)),
        ],
        out_specs=pl.BlockSpec((NUM_GRAPHS, EMB), lambda i: (0, 0)),
        out_shape=jax.ShapeDtypeStruct((NUM_GRAPHS, EMB), jnp.float32),
        scratch_shapes=[
            pltpu.VMEM((NUM_GRAPHS, EMB), jnp.float32),
            pltpu.VMEM((NUM_GRAPHS, EMB), jnp.float32),
        ],
    )(h, batch2d)


def kernel(x, edge_attr, W_edge, b_edge, eps, W1, b1, g1, be1, W2, b2,
           g_bn, b_bn, edge_index, batch):
    src = edge_index[0].astype(jnp.int32)
    dst = edge_index[1].astype(jnp.int32)
    pad = E_PAD - N_EDGES
    src = jnp.pad(src, (0, pad))
    dst = jnp.pad(dst, (0, pad), constant_values=N_NODES)
    attr_pad = jnp.pad(edge_attr, ((0, pad), (0, 0)))
    batch2d = batch.astype(jnp.int32).reshape(N_NODES // PBLK, 1, PBLK)

    h = x
    for l in range(NUM_LAYER):
        e = _encode(attr_pad, W_edge[l], b_edge[l])
        agg2 = _make_sc_aggregate()(h, e, src, dst)
        scale = jnp.full((1, EMB), 1.0 + eps[l], dtype=jnp.float32)
        h = _mlp(h, agg2, scale, W1[l], b1[l], g1[l], be1[l], W2[l], b2[l],
                 g_bn[l], b_bn[l], last=(l == NUM_LAYER - 1))

    return _pool(h, batch2d)


# speedup vs baseline: 2.5926x; 1.9408x over previous
import functools

import jax
import jax.numpy as jnp
from jax import lax
from jax.experimental import pallas as pl
from jax.experimental.pallas import tpu as pltpu
from jax.experimental.pallas import tpu_sc as plsc

NUM_LAYER = 5
EMB = 128
D_EDGE = 16
N_NODES = 10000
N_EDGES = 320000
NUM_GRAPHS = 64

NC = 2
NS = 16
NW = NC * NS

CHUNK = 56
NCHUNK = 180
EW = CHUNK * NCHUNK
E_PAD = EW * NW
NSLOT = 3
NPAD = 10112
ROWS_PER_TILE = NPAD // NS
VEC = 16

@functools.cache
def _make_sc_aggregate():
    mesh = plsc.VectorSubcoreMesh(core_axis_name="c", subcore_axis_name="s",
                                  num_cores=NC, num_subcores=NS)
    return pl.kernel(
        _sc_aggregate_body,
        out_type=jax.ShapeDtypeStruct((NC, NPAD, EMB), jnp.float32),
        mesh=mesh,
        scratch_types=[
            pltpu.VMEM((NSLOT, CHUNK), jnp.int32),
            pltpu.VMEM((NSLOT, CHUNK), jnp.int32),
            pltpu.VMEM((NSLOT, CHUNK, EMB), jnp.float32),
            pltpu.VMEM((NSLOT, CHUNK, EMB), jnp.float32),
            pltpu.VMEM_SHARED((NPAD, EMB), jnp.float32),
            pltpu.SemaphoreType.DMA((NSLOT,)),
            pltpu.SemaphoreType.DMA((NSLOT,)),
            pltpu.SemaphoreType.DMA((NSLOT,)),
            pltpu.SemaphoreType.DMA((NSLOT,)),
            pltpu.SemaphoreType.DMA((NSLOT,)),
        ],
    )


def _sc_aggregate_body(h_hbm, e_hbm, src_hbm, dst_hbm, out_hbm,
                       srcv, dstv, hv, ev, acc,
                       sem_src, sem_dst, sem_e, sem_g, sem_sc):
    c = lax.axis_index("c")
    s = lax.axis_index("s")
    wid = s * NC + c
    edge_base = wid * EW
    row_base = s * ROWS_PER_TILE

    zero = jnp.zeros((VEC,), jnp.float32)

    def _zrow(r, carry):
        for b in range(NSLOT):
            for j in range(EMB // VEC):
                hv[b, r, pl.ds(j * VEC, VEC)] = zero
        return carry
    lax.fori_loop(0, CHUNK, _zrow, 0, unroll=2)

    for t in range(ROWS_PER_TILE // CHUNK):
        pltpu.sync_copy(hv.at[t % NSLOT],
                        acc.at[pl.ds(row_base + t * CHUNK, CHUNK)])
    rem = ROWS_PER_TILE % CHUNK
    pltpu.sync_copy(hv.at[0].at[pl.ds(0, rem)],
                    acc.at[pl.ds(row_base + (ROWS_PER_TILE // CHUNK) * CHUNK,
                                 rem)])
    plsc.subcore_barrier()

    def issue_inputs(k, b):
        base = edge_base + k * CHUNK
        pltpu.async_copy(src_hbm.at[pl.ds(base, CHUNK)], srcv.at[b],
                         sem_src.at[b])
        pltpu.async_copy(dst_hbm.at[pl.ds(base, CHUNK)], dstv.at[b],
                         sem_dst.at[b])
        pltpu.async_copy(e_hbm.at[pl.ds(base, CHUNK)], ev.at[b],
                         sem_e.at[b])

    def wait_src(k, b):
        base = edge_base + k * CHUNK
        pltpu.make_async_copy(src_hbm.at[pl.ds(base, CHUNK)], srcv.at[b],
                              sem_src.at[b]).wait()

    def issue_gather(k, b):
        wait_src(k, b)
        pltpu.async_copy(h_hbm.at[srcv.at[b]], hv.at[b], sem_g.at[b])

    def wait_gather(b):
        pltpu.make_async_copy(h_hbm.at[srcv.at[b]], hv.at[b],
                              sem_g.at[b]).wait()

    def wait_e(k, b):
        base = edge_base + k * CHUNK
        pltpu.make_async_copy(e_hbm.at[pl.ds(base, CHUNK)], ev.at[b],
                              sem_e.at[b]).wait()

    def wait_dst(k, b):
        base = edge_base + k * CHUNK
        pltpu.make_async_copy(dst_hbm.at[pl.ds(base, CHUNK)], dstv.at[b],
                              sem_dst.at[b]).wait()

    def issue_scatter(b):
        pltpu.async_copy(hv.at[b], acc.at[dstv.at[b]], sem_sc.at[b],
                         add=True)

    def drain_scatter(b):
        pltpu.make_async_copy(hv.at[b], acc.at[dstv.at[b]],
                              sem_sc.at[b]).wait()

    def compute(k, b):
        wait_gather(b)
        wait_e(k, b)

        def _crow(r, inner):
            for j in range(EMB // VEC):
                sl = pl.ds(j * VEC, VEC)
                hv[b, r, sl] = jnp.maximum(hv[b, r, sl] + ev[b, r, sl], 0.0)
            return inner
        lax.fori_loop(0, CHUNK, _crow, 0, unroll=4)
        wait_dst(k, b)
        issue_scatter(b)

    issue_inputs(0, 0)
    issue_inputs(1, 1)
    issue_gather(0, 0)

    issue_gather(1, 1)
    compute(0, 0)
    issue_inputs(2, 2)

    def _main(i, carry):
        k0 = 1 + i * NSLOT
        for d in range(NSLOT):
            k = k0 + d
            b = (1 + d) % NSLOT
            bg = (2 + d) % NSLOT
            bi = d % NSLOT
            issue_gather(k + 1, bg)
            compute(k, b)
            drain_scatter(bi)
            issue_inputs(k + 2, bi)
        return carry
    lax.fori_loop(0, (NCHUNK - 3) // NSLOT, _main, 0)

    issue_gather(NCHUNK - 1, (NCHUNK - 1) % NSLOT)
    compute(NCHUNK - 2, (NCHUNK - 2) % NSLOT)
    drain_scatter((NCHUNK - 3) % NSLOT)
    compute(NCHUNK - 1, (NCHUNK - 1) % NSLOT)
    drain_scatter((NCHUNK - 2) % NSLOT)
    drain_scatter((NCHUNK - 1) % NSLOT)

    plsc.subcore_barrier()
    pltpu.sync_copy(acc.at[pl.ds(row_base, ROWS_PER_TILE)],
                    out_hbm.at[c, pl.ds(row_base, ROWS_PER_TILE)])


EBLK = 2048


def _encode_body(attr_ref, w_ref, b_ref, out_ref):
    out_ref[...] = jnp.dot(attr_ref[...], w_ref[...],
                           preferred_element_type=jnp.float32) + b_ref[...]


def _encode(attr_pad, w, b):
    return pl.pallas_call(
        _encode_body,
        grid=(E_PAD // EBLK,),
        in_specs=[
            pl.BlockSpec((EBLK, D_EDGE), lambda i: (i, 0)),
            pl.BlockSpec((D_EDGE, EMB), lambda i: (0, 0)),
            pl.BlockSpec((1, EMB), lambda i: (0, 0)),
        ],
        out_specs=pl.BlockSpec((EBLK, EMB), lambda i: (i, 0)),
        out_shape=jax.ShapeDtypeStruct((E_PAD, EMB), jnp.float32),
    )(attr_pad, w, b.reshape(1, EMB))


NBLK = 400


def _mlp_body(h_ref, a_ref, scale_ref, w1_ref, b1_ref, g1_ref, be1_ref,
              w2_ref, b2_ref, gbn_ref, bbn_ref, out_ref, *, last):
    z = h_ref[...] * scale_ref[...] + a_ref[0] + a_ref[1]
    t = jnp.dot(z, w1_ref[...], preferred_element_type=jnp.float32)
    t = t + b1_ref[...]
    t = jnp.maximum(t * g1_ref[...] + be1_ref[...], 0.0)
    o = jnp.dot(t, w2_ref[...], preferred_element_type=jnp.float32)
    o = o + b2_ref[...]
    o = o * gbn_ref[...] + bbn_ref[...]
    if not last:
        o = jnp.maximum(o, 0.0)
    out_ref[...] = o


def _mlp(h, agg2, scale, w1, b1, g1, be1, w2, b2, gbn, bbn, last):
    body = functools.partial(_mlp_body, last=last)
    row = lambda v: v.reshape(1, -1)
    return pl.pallas_call(
        body,
        grid=(N_NODES // NBLK,),
        in_specs=[
            pl.BlockSpec((NBLK, EMB), lambda i: (i, 0)),
            pl.BlockSpec((NC, NBLK, EMB), lambda i: (0, i, 0)),
            pl.BlockSpec((1, EMB), lambda i: (0, 0)),
            pl.BlockSpec((EMB, 2 * EMB), lambda i: (0, 0)),
            pl.BlockSpec((1, 2 * EMB), lambda i: (0, 0)),
            pl.BlockSpec((1, 2 * EMB), lambda i: (0, 0)),
            pl.BlockSpec((1, 2 * EMB), lambda i: (0, 0)),
            pl.BlockSpec((2 * EMB, EMB), lambda i: (0, 0)),
            pl.BlockSpec((1, EMB), lambda i: (0, 0)),
            pl.BlockSpec((1, EMB), lambda i: (0, 0)),
            pl.BlockSpec((1, EMB), lambda i: (0, 0)),
        ],
        out_specs=pl.BlockSpec((NBLK, EMB), lambda i: (i, 0)),
        out_shape=jax.ShapeDtypeStruct((N_NODES, EMB), jnp.float32),
    )(h, agg2, scale, w1, row(b1), row(g1), row(be1), w2, row(b2),
      row(gbn), row(bbn))


PBLK = 2000


def _pool_body(h_ref, batch_ref, out_ref, sums_ref, cnts_ref):
    i = pl.program_id(0)
    gids = lax.broadcasted_iota(jnp.int32, (NUM_GRAPHS, PBLK), 0)
    oh = (gids == batch_ref[0]).astype(jnp.float32)
    psum = jnp.dot(oh, h_ref[...], preferred_element_type=jnp.float32)
    pcnt = jnp.broadcast_to(jnp.sum(oh, axis=1, keepdims=True),
                            (NUM_GRAPHS, EMB))

    @pl.when(i == 0)
    def _init():
        sums_ref[...] = psum
        cnts_ref[...] = pcnt

    @pl.when(i > 0)
    def _accum():
        sums_ref[...] += psum
        cnts_ref[...] += pcnt

    @pl.when(i == pl.num_programs(0) - 1)
    def _final():
        out_ref[...] = sums_ref[...] / jnp.maximum(cnts_ref[...], 1.0)


def _pool(h, batch2d):
    return pl.pallas_call(
        _pool_body,
        grid=(N_NODES // PBLK,),
        in_specs=[
            pl.BlockSpec((PBLK, EMB), lambda i: (i, 0)),
            pl.BlockSpec((1, 1, PBLK), lambda i: (i, 0, 0)),
        ],
        out_specs=pl.BlockSpec((NUM_GRAPHS, EMB), lambda i: (0, 0)),
        out_shape=jax.ShapeDtypeStruct((NUM_GRAPHS, EMB), jnp.float32),
        scratch_shapes=[
            pltpu.VMEM((NUM_GRAPHS, EMB), jnp.float32),
            pltpu.VMEM((NUM_GRAPHS, EMB), jnp.float32),
        ],
    )(h, batch2d)


def kernel(x, edge_attr, W_edge, b_edge, eps, W1, b1, g1, be1, W2, b2,
           g_bn, b_bn, edge_index, batch):
    src = edge_index[0].astype(jnp.int32)
    dst = edge_index[1].astype(jnp.int32)
    pad = E_PAD - N_EDGES
    src = jnp.pad(src, (0, pad))
    dst = jnp.pad(dst, (0, pad), constant_values=N_NODES)
    attr_pad = jnp.pad(edge_attr, ((0, pad), (0, 0)))
    batch2d = batch.astype(jnp.int32).reshape(N_NODES // PBLK, 1, PBLK)

    h = x
    for l in range(NUM_LAYER):
        e = _encode(attr_pad, W_edge[l], b_edge[l])
        agg2 = _make_sc_aggregate()(h, e, src, dst)
        scale = jnp.full((1, EMB), 1.0 + eps[l], dtype=jnp.float32)
        h = _mlp(h, agg2, scale, W1[l], b1[l], g1[l], be1[l], W2[l], b2[l],
                 g_bn[l], b_bn[l], last=(l == NUM_LAYER - 1))

    return _pool(h, batch2d)


# speedup vs baseline: 2.8452x; 1.0974x over previous
import functools

import jax
import jax.numpy as jnp
from jax import lax
from jax.experimental import pallas as pl
from jax.experimental.pallas import tpu as pltpu
from jax.experimental.pallas import tpu_sc as plsc

NUM_LAYER = 5
EMB = 128
D_EDGE = 16
N_NODES = 10000
N_EDGES = 320000
NUM_GRAPHS = 64

NC = 2
NS = 16
NW = NC * NS

CHUNK = 80
NCHUNK = 126
EW = CHUNK * NCHUNK
E_PAD = EW * NW
NSLOT = 3
NPAD = 10112
ROWS_PER_TILE = NPAD // NS
VEC = 16

@functools.cache
def _make_sc_aggregate():
    mesh = plsc.VectorSubcoreMesh(core_axis_name="c", subcore_axis_name="s",
                                  num_cores=NC, num_subcores=NS)
    return pl.kernel(
        _sc_aggregate_body,
        out_type=jax.ShapeDtypeStruct((NC, NPAD, EMB), jnp.float32),
        mesh=mesh,
        scratch_types=[
            pltpu.VMEM((NSLOT, CHUNK), jnp.int32),
            pltpu.VMEM((NSLOT, CHUNK), jnp.int32),
            pltpu.VMEM((NSLOT, CHUNK, EMB), jnp.float32),
            pltpu.VMEM((CHUNK * EMB // 2,), jnp.int32),
            pltpu.VMEM((CHUNK * EMB // 2,), jnp.int32),
            pltpu.VMEM((CHUNK * EMB // 2,), jnp.int32),
            pltpu.VMEM_SHARED((NPAD, EMB), jnp.float32),
            pltpu.SemaphoreType.DMA((NSLOT,)),
            pltpu.SemaphoreType.DMA((NSLOT,)),
            pltpu.SemaphoreType.DMA((NSLOT,)),
            pltpu.SemaphoreType.DMA((NSLOT,)),
            pltpu.SemaphoreType.DMA((NSLOT,)),
        ],
    )


def _sc_aggregate_body(h_hbm, e_hbm, src_hbm, dst_hbm, out_hbm,
                       srcv, dstv, hv, ev0, ev1, ev2, acc,
                       sem_src, sem_dst, sem_e, sem_g, sem_sc):
    ev = (ev0, ev1, ev2)
    c = lax.axis_index("c")
    s = lax.axis_index("s")
    wid = s * NC + c
    edge_base = wid * EW
    row_base = s * ROWS_PER_TILE

    zero = jnp.zeros((VEC,), jnp.float32)

    def _zrow(r, carry):
        for b in range(NSLOT):
            for j in range(EMB // VEC):
                hv[b, r, pl.ds(j * VEC, VEC)] = zero
        return carry
    lax.fori_loop(0, CHUNK, _zrow, 0, unroll=2)

    for t in range(ROWS_PER_TILE // CHUNK):
        pltpu.sync_copy(hv.at[t % NSLOT],
                        acc.at[pl.ds(row_base + t * CHUNK, CHUNK)])
    rem = ROWS_PER_TILE % CHUNK
    pltpu.sync_copy(hv.at[0].at[pl.ds(0, rem)],
                    acc.at[pl.ds(row_base + (ROWS_PER_TILE // CHUNK) * CHUNK,
                                 rem)])
    plsc.subcore_barrier()

    def issue_inputs(k, b):
        base = edge_base + k * CHUNK
        pltpu.async_copy(src_hbm.at[pl.ds(base, CHUNK)], srcv.at[b],
                         sem_src.at[b])
        pltpu.async_copy(dst_hbm.at[pl.ds(base, CHUNK)], dstv.at[b],
                         sem_dst.at[b])
        pltpu.async_copy(e_hbm.at[pl.ds(base * (EMB // 2), CHUNK * EMB // 2)],
                         ev[b], sem_e.at[b])

    def wait_src(k, b):
        base = edge_base + k * CHUNK
        pltpu.make_async_copy(src_hbm.at[pl.ds(base, CHUNK)], srcv.at[b],
                              sem_src.at[b]).wait()

    def issue_gather(k, b):
        wait_src(k, b)
        pltpu.async_copy(h_hbm.at[srcv.at[b]], hv.at[b], sem_g.at[b])

    def wait_gather(b):
        pltpu.make_async_copy(h_hbm.at[srcv.at[b]], hv.at[b],
                              sem_g.at[b]).wait()

    def wait_e(k, b):
        base = edge_base + k * CHUNK
        pltpu.make_async_copy(
            e_hbm.at[pl.ds(base * (EMB // 2), CHUNK * EMB // 2)],
            ev[b], sem_e.at[b]).wait()

    def wait_dst(k, b):
        base = edge_base + k * CHUNK
        pltpu.make_async_copy(dst_hbm.at[pl.ds(base, CHUNK)], dstv.at[b],
                              sem_dst.at[b]).wait()

    def issue_scatter(b):
        pltpu.async_copy(hv.at[b], acc.at[dstv.at[b]], sem_sc.at[b],
                         add=True)

    def drain_scatter(b):
        pltpu.make_async_copy(hv.at[b], acc.at[dstv.at[b]],
                              sem_sc.at[b]).wait()

    def compute(k, b):
        wait_gather(b)
        wait_e(k, b)

        def _crow(r, inner):
            for j in range(EMB // (2 * VEC)):
                w = ev[b][pl.ds(r * (EMB // 2) + j * VEC, VEC)]
                ea = lax.bitcast_convert_type(w << 16, jnp.float32)
                eo = lax.bitcast_convert_type(w & jnp.int32(-65536),
                                              jnp.float32)
                sl0 = pl.ds(j * 2 * VEC, VEC)
                sl1 = pl.ds(j * 2 * VEC + VEC, VEC)
                hv[b, r, sl0] = jnp.maximum(hv[b, r, sl0] + ea, 0.0)
                hv[b, r, sl1] = jnp.maximum(hv[b, r, sl1] + eo, 0.0)
            return inner
        lax.fori_loop(0, CHUNK, _crow, 0, unroll=4)
        wait_dst(k, b)
        issue_scatter(b)

    issue_inputs(0, 0)
    issue_inputs(1, 1)
    issue_gather(0, 0)

    issue_gather(1, 1)
    compute(0, 0)
    issue_inputs(2, 2)

    def _main(i, carry):
        k0 = 1 + i * NSLOT
        for d in range(NSLOT):
            k = k0 + d
            b = (1 + d) % NSLOT
            bg = (2 + d) % NSLOT
            bi = d % NSLOT
            issue_gather(k + 1, bg)
            compute(k, b)
            drain_scatter(bi)
            issue_inputs(k + 2, bi)
        return carry
    lax.fori_loop(0, (NCHUNK - 3) // NSLOT, _main, 0)

    issue_gather(NCHUNK - 1, (NCHUNK - 1) % NSLOT)
    compute(NCHUNK - 2, (NCHUNK - 2) % NSLOT)
    drain_scatter((NCHUNK - 3) % NSLOT)
    compute(NCHUNK - 1, (NCHUNK - 1) % NSLOT)
    drain_scatter((NCHUNK - 2) % NSLOT)
    drain_scatter((NCHUNK - 1) % NSLOT)

    plsc.subcore_barrier()
    pltpu.sync_copy(acc.at[pl.ds(row_base, ROWS_PER_TILE)],
                    out_hbm.at[c, pl.ds(row_base, ROWS_PER_TILE)])


EBLK = 2048


def _encode_body(attr_ref, w_ref, b_ref, out_ref):
    e = jnp.dot(attr_ref[...], w_ref[...],
                preferred_element_type=jnp.float32) + b_ref[...]
    u = lax.bitcast_convert_type(e, jnp.int32)
    r = (u + jnp.int32(0x7FFF) + ((u >> 16) & 1)) >> 16
    lo = r[:, :EMB // 2] & jnp.int32(0xFFFF)
    hi = r[:, EMB // 2:] << 16
    out_ref[...] = lo | hi


def _encode(attr_pad, w, b):
    return pl.pallas_call(
        _encode_body,
        grid=(E_PAD // EBLK,),
        in_specs=[
            pl.BlockSpec((EBLK, D_EDGE), lambda i: (i, 0)),
            pl.BlockSpec((D_EDGE, EMB), lambda i: (0, 0)),
            pl.BlockSpec((1, EMB), lambda i: (0, 0)),
        ],
        out_specs=pl.BlockSpec((EBLK, EMB // 2), lambda i: (i, 0)),
        out_shape=jax.ShapeDtypeStruct((E_PAD, EMB // 2), jnp.int32),
    )(attr_pad, w, b.reshape(1, EMB))


_EPERM = tuple(
    [32 * (j // VEC) + (j % VEC) for j in range(EMB // 2)]
    + [32 * (j // VEC) + VEC + (j % VEC) for j in range(EMB // 2)]
)


NBLK = 400


def _mlp_body(h_ref, a_ref, scale_ref, w1_ref, b1_ref, g1_ref, be1_ref,
              w2_ref, b2_ref, gbn_ref, bbn_ref, out_ref, *, last):
    z = h_ref[...] * scale_ref[...] + a_ref[0] + a_ref[1]
    t = jnp.dot(z, w1_ref[...], preferred_element_type=jnp.float32)
    t = t + b1_ref[...]
    t = jnp.maximum(t * g1_ref[...] + be1_ref[...], 0.0)
    o = jnp.dot(t, w2_ref[...], preferred_element_type=jnp.float32)
    o = o + b2_ref[...]
    o = o * gbn_ref[...] + bbn_ref[...]
    if not last:
        o = jnp.maximum(o, 0.0)
    out_ref[...] = o


def _mlp(h, agg2, scale, w1, b1, g1, be1, w2, b2, gbn, bbn, last):
    body = functools.partial(_mlp_body, last=last)
    row = lambda v: v.reshape(1, -1)
    return pl.pallas_call(
        body,
        grid=(N_NODES // NBLK,),
        in_specs=[
            pl.BlockSpec((NBLK, EMB), lambda i: (i, 0)),
            pl.BlockSpec((NC, NBLK, EMB), lambda i: (0, i, 0)),
            pl.BlockSpec((1, EMB), lambda i: (0, 0)),
            pl.BlockSpec((EMB, 2 * EMB), lambda i: (0, 0)),
            pl.BlockSpec((1, 2 * EMB), lambda i: (0, 0)),
            pl.BlockSpec((1, 2 * EMB), lambda i: (0, 0)),
            pl.BlockSpec((1, 2 * EMB), lambda i: (0, 0)),
            pl.BlockSpec((2 * EMB, EMB), lambda i: (0, 0)),
            pl.BlockSpec((1, EMB), lambda i: (0, 0)),
            pl.BlockSpec((1, EMB), lambda i: (0, 0)),
            pl.BlockSpec((1, EMB), lambda i: (0, 0)),
        ],
        out_specs=pl.BlockSpec((NBLK, EMB), lambda i: (i, 0)),
        out_shape=jax.ShapeDtypeStruct((N_NODES, EMB), jnp.float32),
    )(h, agg2, scale, w1, row(b1), row(g1), row(be1), w2, row(b2),
      row(gbn), row(bbn))


PBLK = 2000


def _pool_body(h_ref, batch_ref, out_ref, sums_ref, cnts_ref):
    i = pl.program_id(0)
    gids = lax.broadcasted_iota(jnp.int32, (NUM_GRAPHS, PBLK), 0)
    oh = (gids == batch_ref[0]).astype(jnp.float32)
    psum = jnp.dot(oh, h_ref[...], preferred_element_type=jnp.float32)
    pcnt = jnp.broadcast_to(jnp.sum(oh, axis=1, keepdims=True),
                            (NUM_GRAPHS, EMB))

    @pl.when(i == 0)
    def _init():
        sums_ref[...] = psum
        cnts_ref[...] = pcnt

    @pl.when(i > 0)
    def _accum():
        sums_ref[...] += psum
        cnts_ref[...] += pcnt

    @pl.when(i == pl.num_programs(0) - 1)
    def _final():
        out_ref[...] = sums_ref[...] / jnp.maximum(cnts_ref[...], 1.0)


def _pool(h, batch2d):
    return pl.pallas_call(
        _pool_body,
        grid=(N_NODES // PBLK,),
        in_specs=[
            pl.BlockSpec((PBLK, EMB), lambda i: (i, 0)),
            pl.BlockSpec((1, 1, PBLK), lambda i: (i, 0, 0)),
        ],
        out_specs=pl.BlockSpec((NUM_GRAPHS, EMB), lambda i: (0, 0)),
        out_shape=jax.ShapeDtypeStruct((NUM_GRAPHS, EMB), jnp.float32),
        scratch_shapes=[
            pltpu.VMEM((NUM_GRAPHS, EMB), jnp.float32),
            pltpu.VMEM((NUM_GRAPHS, EMB), jnp.float32),
        ],
    )(h, batch2d)


def kernel(x, edge_attr, W_edge, b_edge, eps, W1, b1, g1, be1, W2, b2,
           g_bn, b_bn, edge_index, batch):
    src = edge_index[0].astype(jnp.int32)
    dst = edge_index[1].astype(jnp.int32)
    pad = E_PAD - N_EDGES
    src = jnp.pad(src, (0, pad))
    dst = jnp.pad(dst, (0, pad), constant_values=N_NODES)
    attr_pad = jnp.pad(edge_attr, ((0, pad), (0, 0)))
    batch2d = batch.astype(jnp.int32).reshape(N_NODES // PBLK, 1, PBLK)

    perm = jnp.asarray(_EPERM, dtype=jnp.int32)
    h = x
    for l in range(NUM_LAYER):
        e = _encode(attr_pad, W_edge[l][:, perm],
                    b_edge[l][perm]).reshape(E_PAD * EMB // 2)
        agg2 = _make_sc_aggregate()(h, e, src, dst)
        scale = jnp.full((1, EMB), 1.0 + eps[l], dtype=jnp.float32)
        h = _mlp(h, agg2, scale, W1[l], b1[l], g1[l], be1[l], W2[l], b2[l],
                 g_bn[l], b_bn[l], last=(l == NUM_LAYER - 1))

    return _pool(h, batch2d)
